# baseline (speedup 1.0000x reference)
import functools

import jax
import jax.numpy as jnp
from jax import lax
from jax.experimental import pallas as pl
from jax.experimental.pallas import tpu as pltpu
from jax.experimental.pallas import tpu_sc as plsc

B = 4096
N_SPARSE = 26
D = 64
VOCAB = 100000
NROWS = N_SPARSE * VOCAB

_BLKC = 4096
_OFF = 318 * _BLKC
_NBLK_IN = -(-NROWS // _BLKC) - 1

_NC = 2
_NS = 16
_NW = _NC * _NS
_ROWS = N_SPARSE * B
_RPW = _ROWS // _NW
_CH = 128
_NCHUNK = _RPW // _CH
_NCHUNK_PAD = 32


def _tc_convert(table_t):
    def body(ta, tb, tout):
        a = jnp.transpose(ta[...])
        b = jnp.transpose(tb[...])
        tout[...] = jnp.concatenate([a, b], axis=1)

    return pl.pallas_call(
        body,
        grid=(_OFF // _BLKC,),
        in_specs=[
            pl.BlockSpec((D, _BLKC), lambda g: (0, g)),
            pl.BlockSpec((D, _BLKC), lambda g: (0, jnp.minimum(g + 318, _NBLK_IN))),
        ],
        out_specs=pl.BlockSpec((_BLKC, 128), lambda g: (g, 0)),
        out_shape=jax.ShapeDtypeStruct((_OFF, 128), jnp.float32),
    )(table_t, table_t)


def _sc_gather(table128, ids2d):
    mesh = plsc.VectorSubcoreMesh(core_axis_name="c", subcore_axis_name="s")

    @functools.partial(
        pl.kernel,
        out_type=jax.ShapeDtypeStruct((_ROWS, 128), jnp.float32),
        mesh=mesh,
        scratch_types=[
            pltpu.VMEM((_NCHUNK_PAD, _CH), jnp.int32),
            pltpu.VMEM((_CH, 128), jnp.float32),
            pltpu.SemaphoreType.DMA,
        ],
    )
    def k(table_hbm, ids_hbm, out_hbm, idx_v, buf, sem):
        wid = lax.axis_index("s") * _NC + lax.axis_index("c")
        base = wid * _RPW
        pltpu.sync_copy(ids_hbm.at[pl.ds(wid * _NCHUNK_PAD, _NCHUNK_PAD)], idx_v)

        def body(j, carry):
            pltpu.async_copy(table_hbm.at[idx_v.at[j]], buf, sem).wait()
            pltpu.sync_copy(buf, out_hbm.at[pl.ds(base + j * _CH, _CH)])
            return carry

        lax.fori_loop(0, _NCHUNK, body, 0)

    return k(table128, ids2d)


def _tc_dense(dense_p, emb3, halves_t, W0t, b0c, W1t, b1c, W2t, b2c, T0t, T1t, T2t):
    BLK = 256
    grid = B // BLK

    def body(dx, em, hv, w0, c0, w1, c1, w2, c2, t0, t1, t2, out):
        x = dx[...]
        h = jnp.maximum(
            jnp.dot(w0[...], x, preferred_element_type=jnp.float32) + c0[...], 0.0)
        h = jnp.maximum(
            jnp.dot(w1[...], h, preferred_element_type=jnp.float32) + c1[...], 0.0)
        bot = jnp.maximum(
            jnp.dot(w2[...], h, preferred_element_type=jnp.float32) + c2[...], 0.0)
        hvx = hv[...]
        ts = [bot]
        for i in range(N_SPARSE):
            sel = (hvx[:, i:i + 1] == 1)
            e = jnp.where(sel, em[i][:, D:2 * D], em[i][:, :D])
            ts.append(jnp.transpose(e))
        tstk = jnp.stack(ts, axis=0)
        zs = []
        for i in range(1, N_SPARSE + 1):
            p = tstk[:i] * tstk[i]
            zs.append(jnp.sum(p, axis=1))
        zt = jnp.concatenate(zs, axis=0)
        rt = jnp.concatenate([bot, zt], axis=0)
        h = jnp.maximum(jnp.dot(t0[...], rt, preferred_element_type=jnp.float32), 0.0)
        h = jnp.maximum(jnp.dot(t1[...], h, preferred_element_type=jnp.float32), 0.0)
        out[...] = jnp.dot(t2[...], h, preferred_element_type=jnp.float32)

    full = lambda g: (0, 0)
    return pl.pallas_call(
        body,
        grid=(grid,),
        in_specs=[
            pl.BlockSpec((16, BLK), lambda g: (0, g)),
            pl.BlockSpec((N_SPARSE, BLK, 128), lambda g: (0, g, 0)),
            pl.BlockSpec((BLK, N_SPARSE), lambda g: (g, 0)),
            pl.BlockSpec((512, 16), full),
            pl.BlockSpec((512, 1), full),
            pl.BlockSpec((256, 512), full),
            pl.BlockSpec((256, 1), full),
            pl.BlockSpec((64, 256), full),
            pl.BlockSpec((64, 1), full),
            pl.BlockSpec((512, 415), full),
            pl.BlockSpec((256, 512), full),
            pl.BlockSpec((1, 256), full),
        ],
        out_specs=pl.BlockSpec((1, BLK), lambda g: (0, g)),
        out_shape=jax.ShapeDtypeStruct((1, B), jnp.float32),
    )(dense_p, emb3, halves_t, W0t, b0c, W1t, b1c, W2t, b2c, T0t, T1t, T2t)


def kernel(dense, sparse_ids, emb_table, W0, b0, W1, b1, W2, b2, T0, T1, T2):
    offsets = jnp.arange(N_SPARSE, dtype=sparse_ids.dtype) * VOCAB
    rids = sparse_ids + offsets[None, :]
    halves_t = (rids >= _OFF).astype(jnp.int32)
    pair_ids = jnp.where(rids >= _OFF, rids - _OFF, rids)

    ids_t = pair_ids.T.reshape(_NW, _NCHUNK, _CH)
    ids_pad = (jnp.zeros((_NW, _NCHUNK_PAD, _CH), jnp.int32)
               .at[:, :_NCHUNK].set(ids_t)
               .reshape(_NW * _NCHUNK_PAD, _CH))

    table128 = _tc_convert(emb_table.T)
    emb_flat = _sc_gather(table128, ids_pad)
    emb3 = emb_flat.reshape(N_SPARSE, B, 128)

    dense_p = jnp.zeros((16, B), jnp.float32).at[:13].set(dense.T)
    W0t = jnp.zeros((512, 16), jnp.float32).at[:, :13].set(W0.T)
    out = _tc_dense(
        dense_p, emb3, halves_t,
        W0t, b0.reshape(-1, 1),
        W1.T, b1.reshape(-1, 1),
        W2.T, b2.reshape(-1, 1),
        T0.T, T1.T, T2.T,
    )
    return out.reshape(B)


# speedup vs baseline: 1.3792x; 1.3792x over previous
import functools

import jax
import jax.numpy as jnp
from jax import lax
from jax.experimental import pallas as pl
from jax.experimental.pallas import tpu as pltpu
from jax.experimental.pallas import tpu_sc as plsc

B = 4096
N_SPARSE = 26
D = 64
VOCAB = 100000
NROWS = N_SPARSE * VOCAB

_BLKC = 8192
_NHALF = 159
_OFF = _NHALF * _BLKC
_NBLK_IN = -(-NROWS // _BLKC) - 1

_NC = 2
_NS = 16
_NW = _NC * _NS
_ROWS = N_SPARSE * B
_RPW = _ROWS // _NW
_CH = 128
_NCHUNK = _RPW // _CH
_NCHUNK_PAD = 32


def _tc_convert(table_t):
    def body(ta, tb, tout):
        ab = jnp.concatenate([ta[...], tb[...]], axis=0)
        tout[...] = jnp.transpose(ab)

    return pl.pallas_call(
        body,
        grid=(_OFF // _BLKC,),
        in_specs=[
            pl.BlockSpec((D, _BLKC), lambda g: (0, g)),
            pl.BlockSpec((D, _BLKC),
                         lambda g: (0, jnp.minimum(g + _NHALF, _NBLK_IN))),
        ],
        out_specs=pl.BlockSpec((_BLKC, 128), lambda g: (g, 0)),
        out_shape=jax.ShapeDtypeStruct((_OFF, 128), jnp.float32),
    )(table_t, table_t)


def _sc_gather(table128, ids2d):
    mesh = plsc.VectorSubcoreMesh(core_axis_name="c", subcore_axis_name="s")

    @functools.partial(
        pl.kernel,
        out_type=jax.ShapeDtypeStruct((_ROWS, 128), jnp.float32),
        mesh=mesh,
        scratch_types=[
            pltpu.VMEM((_NCHUNK_PAD, _CH), jnp.int32),
            pltpu.VMEM((_CH, 128), jnp.float32),
            pltpu.SemaphoreType.DMA,
        ],
    )
    def k(table_hbm, ids_hbm, out_hbm, idx_v, buf, sem):
        wid = lax.axis_index("s") * _NC + lax.axis_index("c")
        base = wid * _RPW
        pltpu.sync_copy(ids_hbm.at[pl.ds(wid * _NCHUNK_PAD, _NCHUNK_PAD)], idx_v)

        def body(j, carry):
            pltpu.async_copy(table_hbm.at[idx_v.at[j]], buf, sem).wait()
            pltpu.sync_copy(buf, out_hbm.at[pl.ds(base + j * _CH, _CH)])
            return carry

        lax.fori_loop(0, _NCHUNK, body, 0)

    return k(table128, ids2d)


def _tc_dense(dense_p, emb3, halves_t, W0t, b0c, W1t, b1c, W2t, b2c, T0t, T1t, T2t):
    BLK = 256
    grid = B // BLK

    def body(dx, em, hv, w0, c0, w1, c1, w2, c2, t0, t1, t2, out):
        x = dx[...]
        h = jnp.maximum(
            jnp.dot(w0[...], x, preferred_element_type=jnp.float32) + c0[...], 0.0)
        h = jnp.maximum(
            jnp.dot(w1[...], h, preferred_element_type=jnp.float32) + c1[...], 0.0)
        bot = jnp.maximum(
            jnp.dot(w2[...], h, preferred_element_type=jnp.float32) + c2[...], 0.0)
        hvx = hv[...]
        ts = [bot]
        for i in range(N_SPARSE):
            sel = (hvx[:, i:i + 1] == 1)
            e = jnp.where(sel, em[i][:, D:2 * D], em[i][:, :D])
            ts.append(jnp.transpose(e))
        tstk = jnp.stack(ts, axis=0)
        zs = []
        for i in range(1, N_SPARSE + 1):
            p = tstk[:i] * tstk[i]
            zs.append(jnp.sum(p, axis=1))
        zt = jnp.concatenate(zs, axis=0)
        rt = jnp.concatenate([bot, zt], axis=0)
        h = jnp.maximum(jnp.dot(t0[...], rt, preferred_element_type=jnp.float32), 0.0)
        h = jnp.maximum(jnp.dot(t1[...], h, preferred_element_type=jnp.float32), 0.0)
        out[...] = jnp.dot(t2[...], h, preferred_element_type=jnp.float32)

    full = lambda g: (0, 0)
    return pl.pallas_call(
        body,
        grid=(grid,),
        in_specs=[
            pl.BlockSpec((16, BLK), lambda g: (0, g)),
            pl.BlockSpec((N_SPARSE, BLK, 128), lambda g: (0, g, 0)),
            pl.BlockSpec((BLK, N_SPARSE), lambda g: (g, 0)),
            pl.BlockSpec((512, 16), full),
            pl.BlockSpec((512, 1), full),
            pl.BlockSpec((256, 512), full),
            pl.BlockSpec((256, 1), full),
            pl.BlockSpec((64, 256), full),
            pl.BlockSpec((64, 1), full),
            pl.BlockSpec((512, 415), full),
            pl.BlockSpec((256, 512), full),
            pl.BlockSpec((1, 256), full),
        ],
        out_specs=pl.BlockSpec((1, BLK), lambda g: (0, g)),
        out_shape=jax.ShapeDtypeStruct((1, B), jnp.float32),
    )(dense_p, emb3, halves_t, W0t, b0c, W1t, b1c, W2t, b2c, T0t, T1t, T2t)


def kernel(dense, sparse_ids, emb_table, W0, b0, W1, b1, W2, b2, T0, T1, T2):
    offsets = jnp.arange(N_SPARSE, dtype=sparse_ids.dtype) * VOCAB
    rids = sparse_ids + offsets[None, :]
    halves_t = (rids >= _OFF).astype(jnp.int32)
    pair_ids = jnp.where(rids >= _OFF, rids - _OFF, rids)

    ids_t = pair_ids.T.reshape(_NW, _NCHUNK, _CH)
    ids_pad = (jnp.zeros((_NW, _NCHUNK_PAD, _CH), jnp.int32)
               .at[:, :_NCHUNK].set(ids_t)
               .reshape(_NW * _NCHUNK_PAD, _CH))

    table128 = _tc_convert(emb_table.T)
    emb_flat = _sc_gather(table128, ids_pad)
    emb3 = emb_flat.reshape(N_SPARSE, B, 128)

    dense_p = jnp.zeros((16, B), jnp.float32).at[:13].set(dense.T)
    W0t = jnp.zeros((512, 16), jnp.float32).at[:, :13].set(W0.T)
    out = _tc_dense(
        dense_p, emb3, halves_t,
        W0t, b0.reshape(-1, 1),
        W1.T, b1.reshape(-1, 1),
        W2.T, b2.reshape(-1, 1),
        T0.T, T1.T, T2.T,
    )
    return out.reshape(B)


# speedup vs baseline: 1.4274x; 1.0350x over previous
import functools

import jax
import jax.numpy as jnp
from jax import lax
from jax.experimental import pallas as pl
from jax.experimental.pallas import tpu as pltpu
from jax.experimental.pallas import tpu_sc as plsc

B = 4096
N_SPARSE = 26
D = 64
VOCAB = 100000
NROWS = N_SPARSE * VOCAB

_BLKC = 8192
_NHALF = 159
_OFF = _NHALF * _BLKC
_NBLK_IN = -(-NROWS // _BLKC) - 1

_NC = 2
_NS = 16
_NW = _NC * _NS
_ROWS = N_SPARSE * B
_RPW = _ROWS // _NW
_CH = 128
_NCHUNK = _RPW // _CH
_NCHUNK_PAD = 32


def _tc_convert(table_t):
    def body(ta, tb, tout):
        ab = jnp.concatenate([ta[...], tb[...]], axis=0)
        tout[...] = jnp.transpose(ab)

    return pl.pallas_call(
        body,
        grid=(_OFF // _BLKC,),
        in_specs=[
            pl.BlockSpec((D, _BLKC), lambda g: (0, g)),
            pl.BlockSpec((D, _BLKC),
                         lambda g: (0, jnp.minimum(g + _NHALF, _NBLK_IN))),
        ],
        out_specs=pl.BlockSpec((_BLKC, 128), lambda g: (g, 0)),
        out_shape=jax.ShapeDtypeStruct((_OFF, 128), jnp.float32),
    )(table_t, table_t)


def _sc_gather(table128, ids2d):
    mesh = plsc.VectorSubcoreMesh(core_axis_name="c", subcore_axis_name="s")

    @functools.partial(
        pl.kernel,
        out_type=jax.ShapeDtypeStruct((_ROWS, 128), jnp.float32),
        mesh=mesh,
        scratch_types=[
            pltpu.VMEM((_NCHUNK_PAD, _CH), jnp.int32),
            pltpu.VMEM((_CH, 128), jnp.float32),
            pltpu.VMEM((_CH, 128), jnp.float32),
            pltpu.SemaphoreType.DMA,
            pltpu.SemaphoreType.DMA,
        ],
    )
    def k(table_hbm, ids_hbm, out_hbm, idx_v, buf0, buf1, sem0, sem1):
        wid = lax.axis_index("s") * _NC + lax.axis_index("c")
        base = wid * _RPW
        pltpu.sync_copy(ids_hbm.at[pl.ds(wid * _NCHUNK_PAD, _NCHUNK_PAD)], idx_v)

        first = pltpu.async_copy(table_hbm.at[idx_v.at[0]], buf0, sem0)

        def body(jj, carry):
            j0 = jj * 2
            j1 = j0 + 1

            @pl.when(j1 < _NCHUNK)
            def _():
                pltpu.async_copy(table_hbm.at[idx_v.at[j1]], buf1, sem1)

            pltpu.make_async_copy(table_hbm.at[idx_v.at[j0]], buf0, sem0).wait()
            pltpu.sync_copy(buf0, out_hbm.at[pl.ds(base + j0 * _CH, _CH)])

            @pl.when(j0 + 2 < _NCHUNK)
            def _():
                pltpu.async_copy(table_hbm.at[idx_v.at[j0 + 2]], buf0, sem0)

            @pl.when(j1 < _NCHUNK)
            def _():
                pltpu.make_async_copy(table_hbm.at[idx_v.at[j1]], buf1, sem1).wait()
                pltpu.sync_copy(buf1, out_hbm.at[pl.ds(base + j1 * _CH, _CH)])

            return carry

        lax.fori_loop(0, (_NCHUNK + 1) // 2, body, 0)
        del first

    return k(table128, ids2d)


def _tc_dense(dense_p, emb3, halves_t, W0t, b0c, W1t, b1c, W2t, b2c, T0t, T1t, T2t):
    BLK = 256
    grid = B // BLK

    bf = jnp.bfloat16

    def body(dx, em, hv, w0, c0, w1, c1, w2, c2, t0, t1, t2, out):
        x = dx[...]
        h = jnp.maximum(
            jnp.dot(w0[...], x, preferred_element_type=jnp.float32) + c0[...], 0.0)
        h = jnp.maximum(
            jnp.dot(w1[...], h.astype(bf), preferred_element_type=jnp.float32)
            + c1[...], 0.0)
        bot = jnp.maximum(
            jnp.dot(w2[...], h.astype(bf), preferred_element_type=jnp.float32)
            + c2[...], 0.0)
        hvx = hv[...]
        ts = [bot]
        for i in range(N_SPARSE):
            sel = (hvx[:, i:i + 1] == 1)
            e = jnp.where(sel, em[i][:, D:2 * D], em[i][:, :D])
            ts.append(jnp.transpose(e))
        tstk = jnp.stack(ts, axis=0)
        zs = []
        for i in range(1, N_SPARSE + 1):
            p = tstk[:i] * tstk[i]
            zs.append(jnp.sum(p, axis=1))
        zt = jnp.concatenate(zs, axis=0)
        rt = jnp.concatenate([bot, zt], axis=0).astype(bf)
        h = jnp.maximum(jnp.dot(t0[...], rt, preferred_element_type=jnp.float32), 0.0)
        h = jnp.maximum(
            jnp.dot(t1[...], h.astype(bf), preferred_element_type=jnp.float32), 0.0)
        out[...] = jnp.dot(t2[...], h.astype(bf), preferred_element_type=jnp.float32)

    full = lambda g: (0, 0)
    return pl.pallas_call(
        body,
        grid=(grid,),
        in_specs=[
            pl.BlockSpec((16, BLK), lambda g: (0, g)),
            pl.BlockSpec((N_SPARSE, BLK, 128), lambda g: (0, g, 0)),
            pl.BlockSpec((BLK, N_SPARSE), lambda g: (g, 0)),
            pl.BlockSpec((512, 16), full),
            pl.BlockSpec((512, 1), full),
            pl.BlockSpec((256, 512), full),
            pl.BlockSpec((256, 1), full),
            pl.BlockSpec((64, 256), full),
            pl.BlockSpec((64, 1), full),
            pl.BlockSpec((512, 415), full),
            pl.BlockSpec((256, 512), full),
            pl.BlockSpec((1, 256), full),
        ],
        out_specs=pl.BlockSpec((1, BLK), lambda g: (0, g)),
        out_shape=jax.ShapeDtypeStruct((1, B), jnp.float32),
    )(dense_p, emb3, halves_t, W0t, b0c, W1t, b1c, W2t, b2c, T0t, T1t, T2t)


def kernel(dense, sparse_ids, emb_table, W0, b0, W1, b1, W2, b2, T0, T1, T2):
    offsets = jnp.arange(N_SPARSE, dtype=sparse_ids.dtype) * VOCAB
    rids = sparse_ids + offsets[None, :]
    halves_t = (rids >= _OFF).astype(jnp.int32)
    pair_ids = jnp.where(rids >= _OFF, rids - _OFF, rids)

    ids_t = pair_ids.T.reshape(_NW, _NCHUNK, _CH)
    ids_pad = (jnp.zeros((_NW, _NCHUNK_PAD, _CH), jnp.int32)
               .at[:, :_NCHUNK].set(ids_t)
               .reshape(_NW * _NCHUNK_PAD, _CH))

    table128 = _tc_convert(emb_table.T)
    emb_flat = _sc_gather(table128, ids_pad)
    emb3 = emb_flat.reshape(N_SPARSE, B, 128)

    bf = jnp.bfloat16
    dense_p = jnp.zeros((16, B), bf).at[:13].set(dense.T.astype(bf))
    W0t = jnp.zeros((512, 16), bf).at[:, :13].set(W0.T.astype(bf))
    out = _tc_dense(
        dense_p, emb3, halves_t,
        W0t, b0.reshape(-1, 1),
        W1.T.astype(bf), b1.reshape(-1, 1),
        W2.T.astype(bf), b2.reshape(-1, 1),
        T0.T.astype(bf), T1.T.astype(bf), T2.T.astype(bf),
    )
    return out.reshape(B)


# speedup vs baseline: 1.6712x; 1.1708x over previous
import functools

import jax
import jax.numpy as jnp
from jax import lax
from jax.experimental import pallas as pl
from jax.experimental.pallas import tpu as pltpu
from jax.experimental.pallas import tpu_sc as plsc

B = 4096
N_SPARSE = 26
D = 64
VOCAB = 100000
NROWS = N_SPARSE * VOCAB

_BLKC = 8192
_NQ = 80
_OFF = _NQ * _BLKC
_NBLK_IN = -(-NROWS // _BLKC) - 1

_NC = 2
_NS = 16
_NW = _NC * _NS
_ROWS = N_SPARSE * B
_RPW = _ROWS // _NW
_CH = 128
_NCHUNK = _RPW // _CH
_NCHUNK_PAD = 32


def _tc_convert(table_t):
    def body(ta, tb, tc, td, tout):
        def pack(lo_ref, hi_ref):
            lo = lax.bitcast_convert_type(
                lo_ref[...].astype(jnp.bfloat16), jnp.uint16).astype(jnp.uint32)
            hi = lax.bitcast_convert_type(
                hi_ref[...].astype(jnp.bfloat16), jnp.uint16).astype(jnp.uint32)
            return lo | (hi << 16)

        ab = pack(ta, tb)
        cd = pack(tc, td)
        quad = jnp.concatenate([ab, cd], axis=0)
        tout[...] = lax.bitcast_convert_type(
            jnp.transpose(quad), jnp.float32)

    return pl.pallas_call(
        body,
        grid=(_OFF // _BLKC,),
        in_specs=[
            pl.BlockSpec((D, _BLKC), lambda g: (0, g)),
            pl.BlockSpec((D, _BLKC), lambda g: (0, g + _NQ)),
            pl.BlockSpec((D, _BLKC), lambda g: (0, g + 2 * _NQ)),
            pl.BlockSpec((D, _BLKC),
                         lambda g: (0, jnp.minimum(g + 3 * _NQ, _NBLK_IN))),
        ],
        out_specs=pl.BlockSpec((_BLKC, 128), lambda g: (g, 0)),
        out_shape=jax.ShapeDtypeStruct((_OFF, 128), jnp.float32),
    )(table_t, table_t, table_t, table_t)


def _sc_gather(table128, ids2d):
    mesh = plsc.VectorSubcoreMesh(core_axis_name="c", subcore_axis_name="s")

    @functools.partial(
        pl.kernel,
        out_type=jax.ShapeDtypeStruct((_ROWS, 128), jnp.float32),
        mesh=mesh,
        scratch_types=[
            pltpu.VMEM((_NCHUNK_PAD, _CH), jnp.int32),
            pltpu.VMEM((_CH, 128), jnp.float32),
            pltpu.VMEM((_CH, 128), jnp.float32),
            pltpu.SemaphoreType.DMA,
            pltpu.SemaphoreType.DMA,
        ],
    )
    def k(table_hbm, ids_hbm, out_hbm, idx_v, buf0, buf1, sem0, sem1):
        wid = lax.axis_index("s") * _NC + lax.axis_index("c")
        base = wid * _RPW
        pltpu.sync_copy(ids_hbm.at[pl.ds(wid * _NCHUNK_PAD, _NCHUNK_PAD)], idx_v)

        first = pltpu.async_copy(table_hbm.at[idx_v.at[0]], buf0, sem0)

        def body(jj, carry):
            j0 = jj * 2
            j1 = j0 + 1

            @pl.when(j1 < _NCHUNK)
            def _():
                pltpu.async_copy(table_hbm.at[idx_v.at[j1]], buf1, sem1)

            pltpu.make_async_copy(table_hbm.at[idx_v.at[j0]], buf0, sem0).wait()
            pltpu.sync_copy(buf0, out_hbm.at[pl.ds(base + j0 * _CH, _CH)])

            @pl.when(j0 + 2 < _NCHUNK)
            def _():
                pltpu.async_copy(table_hbm.at[idx_v.at[j0 + 2]], buf0, sem0)

            @pl.when(j1 < _NCHUNK)
            def _():
                pltpu.make_async_copy(table_hbm.at[idx_v.at[j1]], buf1, sem1).wait()
                pltpu.sync_copy(buf1, out_hbm.at[pl.ds(base + j1 * _CH, _CH)])

            return carry

        lax.fori_loop(0, (_NCHUNK + 1) // 2, body, 0)
        del first

    return k(table128, ids2d)


def _tc_dense(dense_p, emb3, halves_t, W0t, b0c, W1t, b1c, W2t, b2c, T0t, T1t, T2t):
    BLK = 256
    grid = B // BLK

    bf = jnp.bfloat16

    def body(dx, em, hv, w0, c0, w1, c1, w2, c2, t0, t1, t2, out):
        x = dx[...]
        h = jnp.maximum(
            jnp.dot(w0[...], x, preferred_element_type=jnp.float32) + c0[...], 0.0)
        h = jnp.maximum(
            jnp.dot(w1[...], h.astype(bf), preferred_element_type=jnp.float32)
            + c1[...], 0.0)
        bot = jnp.maximum(
            jnp.dot(w2[...], h.astype(bf), preferred_element_type=jnp.float32)
            + c2[...], 0.0)
        hvx = hv[...]
        ts = [bot]
        for i in range(N_SPARSE):
            q = hvx[:, i:i + 1]
            u = lax.bitcast_convert_type(em[i], jnp.uint32)
            uh = jnp.where(q >= 2, u[:, D:2 * D], u[:, :D])
            hw = jnp.where((q & 1) == 1, uh >> 16, uh & jnp.uint32(0xFFFF))
            e = lax.bitcast_convert_type(hw << 16, jnp.float32)
            ts.append(jnp.transpose(e))
        tstk = jnp.stack(ts, axis=0)
        zs = []
        for i in range(1, N_SPARSE + 1):
            p = tstk[:i] * tstk[i]
            zs.append(jnp.sum(p, axis=1))
        zt = jnp.concatenate(zs, axis=0)
        rt = jnp.concatenate([bot, zt], axis=0).astype(bf)
        h = jnp.maximum(jnp.dot(t0[...], rt, preferred_element_type=jnp.float32), 0.0)
        h = jnp.maximum(
            jnp.dot(t1[...], h.astype(bf), preferred_element_type=jnp.float32), 0.0)
        out[...] = jnp.dot(t2[...], h.astype(bf), preferred_element_type=jnp.float32)

    full = lambda g: (0, 0)
    return pl.pallas_call(
        body,
        grid=(grid,),
        in_specs=[
            pl.BlockSpec((16, BLK), lambda g: (0, g)),
            pl.BlockSpec((N_SPARSE, BLK, 128), lambda g: (0, g, 0)),
            pl.BlockSpec((BLK, N_SPARSE), lambda g: (g, 0)),
            pl.BlockSpec((512, 16), full),
            pl.BlockSpec((512, 1), full),
            pl.BlockSpec((256, 512), full),
            pl.BlockSpec((256, 1), full),
            pl.BlockSpec((64, 256), full),
            pl.BlockSpec((64, 1), full),
            pl.BlockSpec((512, 415), full),
            pl.BlockSpec((256, 512), full),
            pl.BlockSpec((1, 256), full),
        ],
        out_specs=pl.BlockSpec((1, BLK), lambda g: (0, g)),
        out_shape=jax.ShapeDtypeStruct((1, B), jnp.float32),
    )(dense_p, emb3, halves_t, W0t, b0c, W1t, b1c, W2t, b2c, T0t, T1t, T2t)


def kernel(dense, sparse_ids, emb_table, W0, b0, W1, b1, W2, b2, T0, T1, T2):
    offsets = jnp.arange(N_SPARSE, dtype=sparse_ids.dtype) * VOCAB
    rids = sparse_ids + offsets[None, :]
    halves_t = rids // _OFF
    pair_ids = rids - halves_t * _OFF

    ids_t = pair_ids.T.reshape(_NW, _NCHUNK, _CH)
    ids_pad = (jnp.zeros((_NW, _NCHUNK_PAD, _CH), jnp.int32)
               .at[:, :_NCHUNK].set(ids_t)
               .reshape(_NW * _NCHUNK_PAD, _CH))

    table128 = _tc_convert(emb_table.T)
    emb_flat = _sc_gather(table128, ids_pad)
    emb3 = emb_flat.reshape(N_SPARSE, B, 128)

    bf = jnp.bfloat16
    dense_p = jnp.zeros((16, B), bf).at[:13].set(dense.T.astype(bf))
    W0t = jnp.zeros((512, 16), bf).at[:, :13].set(W0.T.astype(bf))
    out = _tc_dense(
        dense_p, emb3, halves_t,
        W0t, b0.reshape(-1, 1),
        W1.T.astype(bf), b1.reshape(-1, 1),
        W2.T.astype(bf), b2.reshape(-1, 1),
        T0.T.astype(bf), T1.T.astype(bf), T2.T.astype(bf),
    )
    return out.reshape(B)


# speedup vs baseline: 1.7813x; 1.0659x over previous
import functools

import jax
import jax.numpy as jnp
from jax import lax
from jax.experimental import pallas as pl
from jax.experimental.pallas import tpu as pltpu
from jax.experimental.pallas import tpu_sc as plsc

B = 4096
N_SPARSE = 26
D = 64
VOCAB = 100000
NROWS = N_SPARSE * VOCAB

_BLKC = 8192
_NQ = 80
_OFF = _NQ * _BLKC
_NBLK_IN = -(-NROWS // _BLKC) - 1

_NC = 2
_NS = 16
_NW = _NC * _NS
_ROWS = N_SPARSE * B
_RPW = _ROWS // _NW
_CH = 128
_NCHUNK = _RPW // _CH
_NCHUNK_PAD = 32


def _tc_convert(table_t):
    def body(ta, tb, tc, td, tout):
        def pack(lo_ref, hi_ref):
            lo = lax.bitcast_convert_type(
                lo_ref[...].astype(jnp.bfloat16), jnp.uint16).astype(jnp.uint32)
            hi = lax.bitcast_convert_type(
                hi_ref[...].astype(jnp.bfloat16), jnp.uint16).astype(jnp.uint32)
            return lo | (hi << 16)

        ab = pack(ta, tb)
        cd = pack(tc, td)
        quad = jnp.concatenate([ab, cd], axis=0)
        tout[...] = lax.bitcast_convert_type(
            jnp.transpose(quad), jnp.float32)

    return pl.pallas_call(
        body,
        grid=(_OFF // _BLKC,),
        in_specs=[
            pl.BlockSpec((D, _BLKC), lambda g: (0, g)),
            pl.BlockSpec((D, _BLKC), lambda g: (0, g + _NQ)),
            pl.BlockSpec((D, _BLKC), lambda g: (0, g + 2 * _NQ)),
            pl.BlockSpec((D, _BLKC),
                         lambda g: (0, jnp.minimum(g + 3 * _NQ, _NBLK_IN))),
        ],
        out_specs=pl.BlockSpec((_BLKC, 128), lambda g: (g, 0)),
        out_shape=jax.ShapeDtypeStruct((_OFF, 128), jnp.float32),
    )(table_t, table_t, table_t, table_t)


def _sc_gather(table128, ids2d):
    mesh = plsc.VectorSubcoreMesh(core_axis_name="c", subcore_axis_name="s")

    @functools.partial(
        pl.kernel,
        out_type=jax.ShapeDtypeStruct((_ROWS, 128), jnp.float32),
        mesh=mesh,
        scratch_types=[
            pltpu.VMEM((_NCHUNK_PAD, _CH), jnp.int32),
            pltpu.VMEM((_CH, 128), jnp.float32),
            pltpu.VMEM((_CH, 128), jnp.float32),
            pltpu.SemaphoreType.DMA,
            pltpu.SemaphoreType.DMA,
        ],
    )
    def k(table_hbm, ids_hbm, out_hbm, idx_v, buf0, buf1, sem0, sem1):
        wid = lax.axis_index("s") * _NC + lax.axis_index("c")
        base = wid * _RPW
        pltpu.sync_copy(ids_hbm.at[pl.ds(wid * _NCHUNK_PAD, _NCHUNK_PAD)], idx_v)

        first = pltpu.async_copy(table_hbm.at[idx_v.at[0]], buf0, sem0)

        def body(jj, carry):
            j0 = jj * 2
            j1 = j0 + 1

            @pl.when(j1 < _NCHUNK)
            def _():
                pltpu.async_copy(table_hbm.at[idx_v.at[j1]], buf1, sem1)

            pltpu.make_async_copy(table_hbm.at[idx_v.at[j0]], buf0, sem0).wait()
            pltpu.sync_copy(buf0, out_hbm.at[pl.ds(base + j0 * _CH, _CH)])

            @pl.when(j0 + 2 < _NCHUNK)
            def _():
                pltpu.async_copy(table_hbm.at[idx_v.at[j0 + 2]], buf0, sem0)

            @pl.when(j1 < _NCHUNK)
            def _():
                pltpu.make_async_copy(table_hbm.at[idx_v.at[j1]], buf1, sem1).wait()
                pltpu.sync_copy(buf1, out_hbm.at[pl.ds(base + j1 * _CH, _CH)])

            return carry

        lax.fori_loop(0, (_NCHUNK + 1) // 2, body, 0)
        del first

    return k(table128, ids2d)


def _tc_dense(dense_p, emb3, halves_t, W0t, b0c, W1t, b1c, W2t, b2c, T0t, T1t, T2t):
    BLK = 512
    grid = B // BLK

    bf = jnp.bfloat16

    def body(dx, em, hv, w0, c0, w1, c1, w2, c2, t0, t1, t2, out):
        x = dx[...]
        h = jnp.maximum(
            jnp.dot(w0[...], x, preferred_element_type=jnp.float32) + c0[...], 0.0)
        h = jnp.maximum(
            jnp.dot(w1[...], h.astype(bf), preferred_element_type=jnp.float32)
            + c1[...], 0.0)
        bot = jnp.maximum(
            jnp.dot(w2[...], h.astype(bf), preferred_element_type=jnp.float32)
            + c2[...], 0.0)
        hvx = hv[...]
        ts = [bot]
        for i in range(N_SPARSE):
            q = hvx[:, i:i + 1]
            u = lax.bitcast_convert_type(em[i], jnp.uint32)
            uh = jnp.where(q >= 2, u[:, D:2 * D], u[:, :D])
            s = (jnp.uint32(1) - (q.astype(jnp.uint32) & 1)) * 16
            hw = (uh << s) & jnp.uint32(0xFFFF0000)
            e = lax.bitcast_convert_type(hw, jnp.float32)
            ts.append(jnp.transpose(e))
        tstk = jnp.stack(ts, axis=0)
        zs = []
        for i in range(1, N_SPARSE + 1):
            p = tstk[:i] * tstk[i]
            zs.append(jnp.sum(p, axis=1))
        zt = jnp.concatenate(zs, axis=0)
        rt = jnp.concatenate([bot, zt], axis=0).astype(bf)
        h = jnp.maximum(jnp.dot(t0[...], rt, preferred_element_type=jnp.float32), 0.0)
        h = jnp.maximum(
            jnp.dot(t1[...], h.astype(bf), preferred_element_type=jnp.float32), 0.0)
        out[...] = jnp.dot(t2[...], h.astype(bf), preferred_element_type=jnp.float32)

    full = lambda g: (0, 0)
    return pl.pallas_call(
        body,
        grid=(grid,),
        in_specs=[
            pl.BlockSpec((16, BLK), lambda g: (0, g)),
            pl.BlockSpec((N_SPARSE, BLK, 128), lambda g: (0, g, 0)),
            pl.BlockSpec((BLK, N_SPARSE), lambda g: (g, 0)),
            pl.BlockSpec((512, 16), full),
            pl.BlockSpec((512, 1), full),
            pl.BlockSpec((256, 512), full),
            pl.BlockSpec((256, 1), full),
            pl.BlockSpec((64, 256), full),
            pl.BlockSpec((64, 1), full),
            pl.BlockSpec((512, 415), full),
            pl.BlockSpec((256, 512), full),
            pl.BlockSpec((1, 256), full),
        ],
        out_specs=pl.BlockSpec((1, BLK), lambda g: (0, g)),
        out_shape=jax.ShapeDtypeStruct((1, B), jnp.float32),
    )(dense_p, emb3, halves_t, W0t, b0c, W1t, b1c, W2t, b2c, T0t, T1t, T2t)


def kernel(dense, sparse_ids, emb_table, W0, b0, W1, b1, W2, b2, T0, T1, T2):
    offsets = jnp.arange(N_SPARSE, dtype=sparse_ids.dtype) * VOCAB
    rids = sparse_ids + offsets[None, :]
    halves_t = rids // _OFF
    pair_ids = rids - halves_t * _OFF

    ids_t = pair_ids.T.reshape(_NW, _NCHUNK, _CH)
    ids_pad = (jnp.zeros((_NW, _NCHUNK_PAD, _CH), jnp.int32)
               .at[:, :_NCHUNK].set(ids_t)
               .reshape(_NW * _NCHUNK_PAD, _CH))

    table128 = _tc_convert(emb_table.T)
    emb_flat = _sc_gather(table128, ids_pad)
    emb3 = emb_flat.reshape(N_SPARSE, B, 128)

    bf = jnp.bfloat16
    dense_p = jnp.zeros((16, B), bf).at[:13].set(dense.T.astype(bf))
    W0t = jnp.zeros((512, 16), bf).at[:, :13].set(W0.T.astype(bf))
    out = _tc_dense(
        dense_p, emb3, halves_t,
        W0t, b0.reshape(-1, 1),
        W1.T.astype(bf), b1.reshape(-1, 1),
        W2.T.astype(bf), b2.reshape(-1, 1),
        T0.T.astype(bf), T1.T.astype(bf), T2.T.astype(bf),
    )
    return out.reshape(B)


# speedup vs baseline: 1.7924x; 1.0062x over previous
import functools

import jax
import jax.numpy as jnp
from jax import lax
from jax.experimental import pallas as pl
from jax.experimental.pallas import tpu as pltpu
from jax.experimental.pallas import tpu_sc as plsc

B = 4096
N_SPARSE = 26
D = 64
VOCAB = 100000
NROWS = N_SPARSE * VOCAB

_BLKC = 16384
_NQ = 40
_OFF = _NQ * _BLKC
_NBLK_IN = -(-NROWS // _BLKC) - 1

_NC = 2
_NS = 16
_NW = _NC * _NS
_ROWS = N_SPARSE * B
_RPW = _ROWS // _NW
_CH = 128
_NCHUNK = _RPW // _CH
_NCHUNK_PAD = 32


def _tc_convert(table_t):
    def body(ta, tb, tc, td, tout):
        def pack(lo_ref, hi_ref):
            lo = lax.bitcast_convert_type(
                lo_ref[...].astype(jnp.bfloat16), jnp.uint16).astype(jnp.uint32)
            hi = lax.bitcast_convert_type(
                hi_ref[...].astype(jnp.bfloat16), jnp.uint16).astype(jnp.uint32)
            return lo | (hi << 16)

        ab = pack(ta, tb)
        cd = pack(tc, td)
        quad = jnp.concatenate([ab, cd], axis=0)
        tout[...] = lax.bitcast_convert_type(
            jnp.transpose(quad), jnp.float32)

    return pl.pallas_call(
        body,
        grid=(_OFF // _BLKC,),
        in_specs=[
            pl.BlockSpec((D, _BLKC), lambda g: (0, g)),
            pl.BlockSpec((D, _BLKC), lambda g: (0, g + _NQ)),
            pl.BlockSpec((D, _BLKC), lambda g: (0, g + 2 * _NQ)),
            pl.BlockSpec((D, _BLKC),
                         lambda g: (0, jnp.minimum(g + 3 * _NQ, _NBLK_IN))),
        ],
        out_specs=pl.BlockSpec((_BLKC, 128), lambda g: (g, 0)),
        out_shape=jax.ShapeDtypeStruct((_OFF, 128), jnp.float32),
    )(table_t, table_t, table_t, table_t)


def _sc_gather(table128, ids2d):
    mesh = plsc.VectorSubcoreMesh(core_axis_name="c", subcore_axis_name="s")

    @functools.partial(
        pl.kernel,
        out_type=jax.ShapeDtypeStruct((_ROWS, 128), jnp.float32),
        mesh=mesh,
        scratch_types=[
            pltpu.VMEM((_NCHUNK_PAD, _CH), jnp.int32),
            pltpu.VMEM((_CH, 128), jnp.float32),
            pltpu.VMEM((_CH, 128), jnp.float32),
            pltpu.SemaphoreType.DMA,
            pltpu.SemaphoreType.DMA,
        ],
    )
    def k(table_hbm, ids_hbm, out_hbm, idx_v, buf0, buf1, sem0, sem1):
        wid = lax.axis_index("s") * _NC + lax.axis_index("c")
        base = wid * _RPW
        pltpu.sync_copy(ids_hbm.at[pl.ds(wid * _NCHUNK_PAD, _NCHUNK_PAD)], idx_v)

        first = pltpu.async_copy(table_hbm.at[idx_v.at[0]], buf0, sem0)

        def body(jj, carry):
            j0 = jj * 2
            j1 = j0 + 1

            @pl.when(j1 < _NCHUNK)
            def _():
                pltpu.async_copy(table_hbm.at[idx_v.at[j1]], buf1, sem1)

            pltpu.make_async_copy(table_hbm.at[idx_v.at[j0]], buf0, sem0).wait()
            pltpu.sync_copy(buf0, out_hbm.at[pl.ds(base + j0 * _CH, _CH)])

            @pl.when(j0 + 2 < _NCHUNK)
            def _():
                pltpu.async_copy(table_hbm.at[idx_v.at[j0 + 2]], buf0, sem0)

            @pl.when(j1 < _NCHUNK)
            def _():
                pltpu.make_async_copy(table_hbm.at[idx_v.at[j1]], buf1, sem1).wait()
                pltpu.sync_copy(buf1, out_hbm.at[pl.ds(base + j1 * _CH, _CH)])

            return carry

        lax.fori_loop(0, (_NCHUNK + 1) // 2, body, 0)
        del first

    return k(table128, ids2d)


def _tc_dense(dense_p, emb3, halves_t, W0t, b0c, W1t, b1c, W2t, b2c, T0t, T1t, T2t):
    BLK = 512
    grid = B // BLK

    bf = jnp.bfloat16

    def body(dx, em, hv, w0, c0, w1, c1, w2, c2, t0, t1, t2, out):
        x = dx[...]
        h = jnp.maximum(
            jnp.dot(w0[...], x, preferred_element_type=jnp.float32) + c0[...], 0.0)
        h = jnp.maximum(
            jnp.dot(w1[...], h.astype(bf), preferred_element_type=jnp.float32)
            + c1[...], 0.0)
        bot = jnp.maximum(
            jnp.dot(w2[...], h.astype(bf), preferred_element_type=jnp.float32)
            + c2[...], 0.0)
        hvx = hv[...]
        ts = [bot]
        for i in range(N_SPARSE):
            q = hvx[:, i:i + 1]
            u = lax.bitcast_convert_type(em[i], jnp.uint32)
            uh = jnp.where(q >= 2, u[:, D:2 * D], u[:, :D])
            s = (jnp.uint32(1) - (q.astype(jnp.uint32) & 1)) * 16
            hw = (uh << s) & jnp.uint32(0xFFFF0000)
            e = lax.bitcast_convert_type(hw, jnp.float32)
            ts.append(jnp.transpose(e))
        tstk = jnp.stack(ts, axis=0)
        zs = []
        for i in range(1, N_SPARSE + 1):
            p = tstk[:i] * tstk[i]
            zs.append(jnp.sum(p, axis=1))
        zt = jnp.concatenate(zs, axis=0)
        rt = jnp.concatenate([bot, zt], axis=0).astype(bf)
        h = jnp.maximum(jnp.dot(t0[...], rt, preferred_element_type=jnp.float32), 0.0)
        h = jnp.maximum(
            jnp.dot(t1[...], h.astype(bf), preferred_element_type=jnp.float32), 0.0)
        out[...] = jnp.dot(t2[...], h.astype(bf), preferred_element_type=jnp.float32)

    full = lambda g: (0, 0)
    return pl.pallas_call(
        body,
        grid=(grid,),
        in_specs=[
            pl.BlockSpec((16, BLK), lambda g: (0, g)),
            pl.BlockSpec((N_SPARSE, BLK, 128), lambda g: (0, g, 0)),
            pl.BlockSpec((BLK, N_SPARSE), lambda g: (g, 0)),
            pl.BlockSpec((512, 16), full),
            pl.BlockSpec((512, 1), full),
            pl.BlockSpec((256, 512), full),
            pl.BlockSpec((256, 1), full),
            pl.BlockSpec((64, 256), full),
            pl.BlockSpec((64, 1), full),
            pl.BlockSpec((512, 415), full),
            pl.BlockSpec((256, 512), full),
            pl.BlockSpec((1, 256), full),
        ],
        out_specs=pl.BlockSpec((1, BLK), lambda g: (0, g)),
        out_shape=jax.ShapeDtypeStruct((1, B), jnp.float32),
    )(dense_p, emb3, halves_t, W0t, b0c, W1t, b1c, W2t, b2c, T0t, T1t, T2t)


def kernel(dense, sparse_ids, emb_table, W0, b0, W1, b1, W2, b2, T0, T1, T2):
    offsets = jnp.arange(N_SPARSE, dtype=sparse_ids.dtype) * VOCAB
    rids = sparse_ids + offsets[None, :]
    halves_t = rids // _OFF
    pair_ids = rids - halves_t * _OFF

    ids_t = pair_ids.T.reshape(_NW, _NCHUNK, _CH)
    ids_pad = (jnp.zeros((_NW, _NCHUNK_PAD, _CH), jnp.int32)
               .at[:, :_NCHUNK].set(ids_t)
               .reshape(_NW * _NCHUNK_PAD, _CH))

    table128 = _tc_convert(emb_table.T)
    emb_flat = _sc_gather(table128, ids_pad)
    emb3 = emb_flat.reshape(N_SPARSE, B, 128)

    bf = jnp.bfloat16
    dense_p = jnp.zeros((16, B), bf).at[:13].set(dense.T.astype(bf))
    W0t = jnp.zeros((512, 16), bf).at[:, :13].set(W0.T.astype(bf))
    out = _tc_dense(
        dense_p, emb3, halves_t,
        W0t, b0.reshape(-1, 1),
        W1.T.astype(bf), b1.reshape(-1, 1),
        W2.T.astype(bf), b2.reshape(-1, 1),
        T0.T.astype(bf), T1.T.astype(bf), T2.T.astype(bf),
    )
    return out.reshape(B)
